# baseline (speedup 1.0000x reference)
import jax
import jax.numpy as jnp
from jax import lax
from jax.experimental import pallas as pl
from jax.experimental.pallas import tpu as pltpu
from jax.experimental.pallas import tpu_sc as plsc

_S = 30.0
_M = 0.5
_NC = 2
_NS = 16
_NW = _NC * _NS
_L = 16
_ROWS = 1000
_COLS = 16384


def _sc_body(x_hbm, lab_hbm, out_hbm,
             in0, in1, o0, o1, labv, si0, si1, so0, so1):
    wid = lax.axis_index("s") * _NC + lax.axis_index("c")
    pltpu.sync_copy(lab_hbm, labv)
    ins, outs = (in0, in1), (o0, o1)
    sin, sout = (si0, si1), (so0, so1)

    t0 = wid
    @pl.when(t0 < _ROWS)
    def _():
        pltpu.async_copy(x_hbm.at[pl.ds(t0, 1)], ins[0], sin[0])

    @pl.loop(0, 16)
    def _pair(p):
        for b in range(2):
            m = p * 2 + b
            t = wid + m * _NW
            nxt = t + _NW

            @pl.when(nxt < _ROWS)
            def _():
                pltpu.async_copy(x_hbm.at[pl.ds(nxt, 1)], ins[1 - b],
                                 sin[1 - b])

            @pl.when(t < _ROWS)
            def _():
                pltpu.make_async_copy(x_hbm.at[pl.ds(t, 1)], ins[b],
                                      sin[b]).wait()

                @pl.when(m >= 2)
                def _():
                    pltpu.make_async_copy(
                        outs[b], out_hbm.at[pl.ds(t - 2 * _NW, 1)],
                        sout[b]).wait()

                @pl.loop(0, _COLS // _L, unroll=4)
                def _col(j):
                    col = j * _L
                    lab = labv[pl.ds(col, _L)]
                    x = ins[b][0, pl.ds(col, _L)]
                    y = jnp.where(lab >= 0, x * (1.0 - _S) + (_S * _M), 0.0)
                    outs[b][0, pl.ds(col, _L)] = y

                pltpu.async_copy(outs[b], out_hbm.at[pl.ds(t, 1)], sout[b])

    t_b0 = wid + 30 * _NW
    pltpu.make_async_copy(outs[0], out_hbm.at[pl.ds(t_b0, 1)], sout[0]).wait()
    t_b1_hi = wid + 31 * _NW
    t_b1_lo = wid + 29 * _NW

    @pl.when(t_b1_hi < _ROWS)
    def _():
        pltpu.make_async_copy(outs[1], out_hbm.at[pl.ds(t_b1_hi, 1)],
                              sout[1]).wait()

    @pl.when(t_b1_hi >= _ROWS)
    def _():
        pltpu.make_async_copy(outs[1], out_hbm.at[pl.ds(t_b1_lo, 1)],
                              sout[1]).wait()


def kernel(projected, labels, W):
    del W
    xt = projected.T
    out_t = pl.kernel(
        _sc_body,
        out_type=jax.ShapeDtypeStruct((_ROWS, _COLS), jnp.float32),
        mesh=plsc.VectorSubcoreMesh(core_axis_name="c", subcore_axis_name="s",
                                    num_cores=_NC, num_subcores=_NS),
        scratch_types=[
            pltpu.VMEM((1, _COLS), jnp.float32),
            pltpu.VMEM((1, _COLS), jnp.float32),
            pltpu.VMEM((1, _COLS), jnp.float32),
            pltpu.VMEM((1, _COLS), jnp.float32),
            pltpu.VMEM((_COLS,), jnp.int32),
            pltpu.SemaphoreType.DMA,
            pltpu.SemaphoreType.DMA,
            pltpu.SemaphoreType.DMA,
            pltpu.SemaphoreType.DMA,
        ],
    )(xt, labels)
    return out_t.T


# speedup vs baseline: 3.0931x; 3.0931x over previous
import jax
import jax.numpy as jnp
from jax import lax
from jax.experimental import pallas as pl
from jax.experimental.pallas import tpu as pltpu
from jax.experimental.pallas import tpu_sc as plsc

_S = 30.0
_M = 0.5
_NC = 2
_NS = 16
_NW = _NC * _NS
_L = 16
_ROWS = 1000
_COLS = 16384


def _sc_body(x_hbm, lab_hbm, out_hbm,
             in0, in1, o0, o1, labv, si0, si1, so0, so1):
    wid = lax.axis_index("s") * _NC + lax.axis_index("c")
    pltpu.sync_copy(lab_hbm, labv)
    ins, outs = (in0, in1), (o0, o1)
    sin, sout = (si0, si1), (so0, so1)

    t0 = wid
    @pl.when(t0 < _ROWS)
    def _():
        pltpu.async_copy(x_hbm.at[pl.ds(t0, 1)], ins[0], sin[0])

    @pl.loop(0, 16)
    def _pair(p):
        for b in range(2):
            m = p * 2 + b
            t = wid + m * _NW
            nxt = t + _NW

            @pl.when(nxt < _ROWS)
            def _():
                pltpu.async_copy(x_hbm.at[pl.ds(nxt, 1)], ins[1 - b],
                                 sin[1 - b])

            @pl.when(t < _ROWS)
            def _():
                pltpu.make_async_copy(x_hbm.at[pl.ds(t, 1)], ins[b],
                                      sin[b]).wait()

                @pl.when(m >= 2)
                def _():
                    pltpu.make_async_copy(
                        outs[b], out_hbm.at[pl.ds(t - 2 * _NW, 1)],
                        sout[b]).wait()

                @plsc.parallel_loop(0, _COLS // _L, unroll=8)
                def _col(j):
                    col = j * _L
                    lab = labv[pl.ds(col, _L)]
                    x = ins[b][0, pl.ds(col, _L)]
                    y = jnp.where(lab >= 0, x * (1.0 - _S) + (_S * _M), 0.0)
                    outs[b][0, pl.ds(col, _L)] = y

                pltpu.async_copy(outs[b], out_hbm.at[pl.ds(t, 1)], sout[b])

    t_b0 = wid + 30 * _NW
    pltpu.make_async_copy(outs[0], out_hbm.at[pl.ds(t_b0, 1)], sout[0]).wait()
    t_b1_hi = wid + 31 * _NW
    t_b1_lo = wid + 29 * _NW

    @pl.when(t_b1_hi < _ROWS)
    def _():
        pltpu.make_async_copy(outs[1], out_hbm.at[pl.ds(t_b1_hi, 1)],
                              sout[1]).wait()

    @pl.when(t_b1_hi >= _ROWS)
    def _():
        pltpu.make_async_copy(outs[1], out_hbm.at[pl.ds(t_b1_lo, 1)],
                              sout[1]).wait()


def kernel(projected, labels, W):
    del W
    xt = projected.T
    out_t = pl.kernel(
        _sc_body,
        out_type=jax.ShapeDtypeStruct((_ROWS, _COLS), jnp.float32),
        mesh=plsc.VectorSubcoreMesh(core_axis_name="c", subcore_axis_name="s",
                                    num_cores=_NC, num_subcores=_NS),
        scratch_types=[
            pltpu.VMEM((1, _COLS), jnp.float32),
            pltpu.VMEM((1, _COLS), jnp.float32),
            pltpu.VMEM((1, _COLS), jnp.float32),
            pltpu.VMEM((1, _COLS), jnp.float32),
            pltpu.VMEM((_COLS,), jnp.int32),
            pltpu.SemaphoreType.DMA,
            pltpu.SemaphoreType.DMA,
            pltpu.SemaphoreType.DMA,
            pltpu.SemaphoreType.DMA,
        ],
    )(xt, labels)
    return out_t.T


# speedup vs baseline: 3.3568x; 1.0852x over previous
import jax
import jax.numpy as jnp
from jax import lax
from jax.experimental import pallas as pl
from jax.experimental.pallas import tpu as pltpu
from jax.experimental.pallas import tpu_sc as plsc

_S = 30.0
_M = 0.5
_NC = 2
_NS = 16
_NW = _NC * _NS
_L = 16
_ROWS = 1000
_COLS = 16384


def _sc_body(x_hbm, lab_hbm, out_hbm,
             in0, in1, o0, o1, labv, si0, si1, so0, so1):
    wid = lax.axis_index("s") * _NC + lax.axis_index("c")
    pltpu.sync_copy(lab_hbm, labv)
    ins, outs = (in0, in1), (o0, o1)
    sin, sout = (si0, si1), (so0, so1)

    t0 = wid
    @pl.when(t0 < _ROWS)
    def _():
        pltpu.async_copy(x_hbm.at[pl.ds(t0, 1)], ins[0], sin[0])

    @pl.loop(0, 16)
    def _pair(p):
        for b in range(2):
            m = p * 2 + b
            t = wid + m * _NW
            nxt = t + _NW

            @pl.when(nxt < _ROWS)
            def _():
                pltpu.async_copy(x_hbm.at[pl.ds(nxt, 1)], ins[1 - b],
                                 sin[1 - b])

            @pl.when(t < _ROWS)
            def _():
                pltpu.make_async_copy(x_hbm.at[pl.ds(t, 1)], ins[b],
                                      sin[b]).wait()

                @pl.when(m >= 2)
                def _():
                    pltpu.make_async_copy(
                        outs[b], out_hbm.at[pl.ds(t - 2 * _NW, 1)],
                        sout[b]).wait()

                @plsc.parallel_loop(0, _COLS // _L, unroll=8)
                def _col(j):
                    col = j * _L
                    x = ins[b][0, pl.ds(col, _L)]
                    outs[b][0, pl.ds(col, _L)] = x * (1.0 - _S) + (_S * _M)

                pltpu.async_copy(outs[b], out_hbm.at[pl.ds(t, 1)], sout[b])

    t_b0 = wid + 30 * _NW
    pltpu.make_async_copy(outs[0], out_hbm.at[pl.ds(t_b0, 1)], sout[0]).wait()
    t_b1_hi = wid + 31 * _NW
    t_b1_lo = wid + 29 * _NW

    @pl.when(t_b1_hi < _ROWS)
    def _():
        pltpu.make_async_copy(outs[1], out_hbm.at[pl.ds(t_b1_hi, 1)],
                              sout[1]).wait()

    @pl.when(t_b1_hi >= _ROWS)
    def _():
        pltpu.make_async_copy(outs[1], out_hbm.at[pl.ds(t_b1_lo, 1)],
                              sout[1]).wait()


def kernel(projected, labels, W):
    del W
    xt = projected.T
    out_t = pl.kernel(
        _sc_body,
        out_type=jax.ShapeDtypeStruct((_ROWS, _COLS), jnp.float32),
        mesh=plsc.VectorSubcoreMesh(core_axis_name="c", subcore_axis_name="s",
                                    num_cores=_NC, num_subcores=_NS),
        scratch_types=[
            pltpu.VMEM((1, _COLS), jnp.float32),
            pltpu.VMEM((1, _COLS), jnp.float32),
            pltpu.VMEM((1, _COLS), jnp.float32),
            pltpu.VMEM((1, _COLS), jnp.float32),
            pltpu.VMEM((_COLS,), jnp.int32),
            pltpu.SemaphoreType.DMA,
            pltpu.SemaphoreType.DMA,
            pltpu.SemaphoreType.DMA,
            pltpu.SemaphoreType.DMA,
        ],
    )(xt, labels)
    return out_t.T


# speedup vs baseline: 5.7550x; 1.7144x over previous
import jax
import jax.numpy as jnp
from jax.experimental import pallas as pl

_S = 30.0
_M = 0.5
_BLOCK_R = 200


def _arcface_block(lab_ref, x_ref, o_ref):
    keep = lab_ref[...] >= 0
    o_ref[...] = jnp.where(keep, x_ref[...] * (1.0 - _S) + (_S * _M), 0.0)


def kernel(projected, labels, W):
    del W
    batch, out_f = projected.shape
    xt = projected.T
    lab = labels.reshape(1, batch)
    grid = (out_f // _BLOCK_R,)
    out_t = pl.pallas_call(
        _arcface_block,
        grid=grid,
        in_specs=[
            pl.BlockSpec((1, batch), lambda i: (0, 0)),
            pl.BlockSpec((_BLOCK_R, batch), lambda i: (i, 0)),
        ],
        out_specs=pl.BlockSpec((_BLOCK_R, batch), lambda i: (i, 0)),
        out_shape=jax.ShapeDtypeStruct((out_f, batch), projected.dtype),
    )(lab, xt)
    return out_t.T
